# Initial kernel scaffold; baseline (speedup 1.0000x reference)
import jax
import jax.numpy as jnp
from jax.experimental import pallas as pl


def kernel(feature, edge_index, W, b):
    raise NotImplementedError("write your pallas kernel here")


# speedup vs baseline: 4.5508x; 4.5508x over previous
import functools

import jax
import jax.numpy as jnp
from jax import lax
from jax.experimental import pallas as pl
from jax.experimental.pallas import tpu as pltpu
from jax.experimental.pallas import tpu_sc as plsc

N_NODES_K = 10000
D_K = 128
ACC_ROWS = 10240
CHUNK = 128
NC, NS = 2, 16
NW = NC * NS


def _sc_aggregate(feature, edge_pairs, n_chunks_per_worker):
    mesh = plsc.VectorSubcoreMesh(core_axis_name="c", subcore_axis_name="s")

    @functools.partial(
        pl.kernel,
        out_type=jax.ShapeDtypeStruct((NC, N_NODES_K, D_K), jnp.float32),
        mesh=mesh,
        scratch_types=[
            pltpu.VMEM((2, CHUNK), jnp.int32),
            pltpu.VMEM((CHUNK, D_K), jnp.float32),
            pltpu.VMEM((CHUNK, D_K), jnp.float32),
            pltpu.VMEM_SHARED((ACC_ROWS, D_K), jnp.float32),
            pltpu.SemaphoreType.DMA,
        ],
    )
    def k(feat_hbm, pairs_hbm, out_hbm, idx_v, rows_v, zeros_v, acc_s, sem):
        core = lax.axis_index("c")
        sid = lax.axis_index("s")
        wid = sid * NC + core

        @pl.loop(0, CHUNK)
        def _(r):
            @pl.loop(0, D_K, step=16)
            def _(c0):
                zeros_v[r, pl.ds(c0, 16)] = jnp.zeros((16,), jnp.float32)

        stripe = ACC_ROWS // NS
        @pl.loop(0, stripe, step=CHUNK)
        def _(z):
            pltpu.sync_copy(zeros_v, acc_s.at[pl.ds(sid * stripe + z, CHUNK)])

        plsc.subcore_barrier()

        @pl.loop(0, n_chunks_per_worker)
        def _(j):
            cid = wid * n_chunks_per_worker + j
            pltpu.sync_copy(pairs_hbm.at[cid], idx_v)
            pltpu.async_copy(feat_hbm.at[idx_v.at[0]], rows_v, sem).wait()
            pltpu.sync_copy(rows_v, acc_s.at[idx_v.at[1]], add=True)

        plsc.subcore_barrier()

        @pl.when(sid < NS - 1)
        def _():
            pltpu.sync_copy(
                acc_s.at[pl.ds(sid * stripe, stripe)],
                out_hbm.at[core].at[pl.ds(sid * stripe, stripe)],
            )

        @pl.when(sid == NS - 1)
        def _():
            last = N_NODES_K - (NS - 1) * stripe
            pltpu.sync_copy(
                acc_s.at[pl.ds((NS - 1) * stripe, last)],
                out_hbm.at[core].at[pl.ds((NS - 1) * stripe, last)],
            )

    return k(feature, edge_pairs)


def _tc_body(p_ref, w_ref, b_ref, o_ref):
    agg = p_ref[0] + p_ref[1]
    h = jnp.dot(agg, w_ref[...], preferred_element_type=jnp.float32)
    o_ref[...] = jnp.maximum(h + b_ref[...], 0.0)


def _tc_apply(partials, W, b):
    blk = 2000
    return pl.pallas_call(
        _tc_body,
        grid=(N_NODES_K // blk,),
        in_specs=[
            pl.BlockSpec((NC, blk, D_K), lambda i: (0, i, 0)),
            pl.BlockSpec((D_K, D_K), lambda i: (0, 0)),
            pl.BlockSpec((1, D_K), lambda i: (0, 0)),
        ],
        out_specs=pl.BlockSpec((blk, D_K), lambda i: (i, 0)),
        out_shape=jax.ShapeDtypeStruct((N_NODES_K, D_K), jnp.float32),
    )(partials, W, b.reshape(1, D_K))


def kernel(feature, edge_index, W, b):
    e = edge_index.shape[1]
    epad = ((e + NW * CHUNK - 1) // (NW * CHUNK)) * (NW * CHUNK)
    pad = epad - e
    src = jnp.concatenate([edge_index[0], jnp.zeros((pad,), jnp.int32)])
    dst = jnp.concatenate(
        [edge_index[1], jnp.full((pad,), N_NODES_K, jnp.int32)]
    )
    pairs = jnp.stack(
        [src.reshape(-1, CHUNK), dst.reshape(-1, CHUNK)], axis=1
    )
    partials = _sc_aggregate(feature, pairs, epad // (NW * CHUNK))
    return _tc_apply(partials, W, b)
